# Initial kernel scaffold; baseline (speedup 1.0000x reference)
import jax
import jax.numpy as jnp
from jax.experimental import pallas as pl


def kernel(target, contex, embed_target_w, embed_contex_w, proto_memory, repe_end):
    raise NotImplementedError("write your pallas kernel here")


# speedup vs baseline: 2.3691x; 2.3691x over previous
import functools

import jax
import jax.numpy as jnp
from jax import lax
from jax.experimental import pallas as pl
from jax.experimental.pallas import tpu as pltpu
from jax.experimental.pallas import tpu_sc as plsc

B = 4096
LC = 50
LCP = 56
D = 64
M = 100000
K = 32
SCALE = 8.0

NC, NS = 2, 16
NW = NC * NS
RPW = B // NW
CPD = 2
NCHUNK = RPW // CPD
IPC = CPD * LCP
NBLK = 100
RBLK = M // NBLK


def _attend_kernel(cpad, tgtids, ecw, etw,
                   pc_out, uv_out, t0_out,
                   cidx_v, tidx_v, t0i_v, ctx_v, tgt_v, t0r_v, uv_v, pc_v,
                   sem_t, sem0, sem1):
    wid = lax.axis_index("s") * NC + lax.axis_index("c")
    base = wid * RPW

    pltpu.sync_copy(cpad.at[pl.ds(base * LCP, RPW * LCP)], cidx_v)
    pltpu.sync_copy(tgtids.at[pl.ds(base, RPW)], tidx_v)
    pltpu.sync_copy(tgtids.at[pl.ds(0, 8)], t0i_v)
    pltpu.async_copy(etw.at[tidx_v], tgt_v, sem_t).wait()
    pltpu.async_copy(etw.at[t0i_v], t0r_v, sem_t).wait()

    t0v = [t0r_v[0, pl.ds(l * 16, 16)] for l in range(4)]
    zero = jnp.zeros((16,), jnp.float32)
    for l in range(4):
        pc_v[0, pl.ds(l * 16, 16)] = zero

    sems = (sem0, sem1)

    def issue(j, slot):
        off = pl.multiple_of(j * IPC, 8)
        return pltpu.async_copy(
            ecw.at[cidx_v.at[pl.ds(off, IPC)]], ctx_v.at[slot], sems[slot])

    def drain(slot):
        pltpu.make_async_copy(
            ecw.at[cidx_v.at[pl.ds(0, IPC)]], ctx_v.at[slot], sems[slot]).wait()

    issue(0, 0)
    lane = lax.broadcasted_iota(jnp.int32, (16,), 0)

    def bf16r(x):
        i = plsc.bitcast(x, jnp.int32)
        lsb = lax.shift_right_logical(i, 16) & jnp.int32(1)
        i = i + jnp.int32(0x7FFF) + lsb
        i = i & jnp.int32(-65536)
        return plsc.bitcast(i, jnp.float32)

    @pl.loop(0, NCHUNK // 2, init_carry=(zero, zero))
    def _chunks(g, carry):
        u16, v16 = carry
        for slot in range(2):
            j = g * 2 + slot
            drain(slot)

            @pl.when(j + 1 < NCHUNK)
            def _():
                issue(j + 1, 1 - slot)

            for s2 in range(CPD):
                jj = j * CPD + s2
                tl = [tgt_v[jj, pl.ds(l * 16, 16)] for l in range(4)]

                svecs = [zero, zero, zero, zero]
                mrow = jnp.float32(-jnp.inf)
                for c in range(LC):
                    x = [ctx_v[slot, s2 * LCP + c, pl.ds(l * 16, 16)]
                         for l in range(4)]
                    p = (x[0] * tl[0] + x[1] * tl[1]
                         + x[2] * tl[2] + x[3] * tl[3])
                    s = jnp.sum(p)
                    svecs[c // 16] = jnp.where(lane == (c % 16), s,
                                               svecs[c // 16])
                    mrow = jnp.maximum(mrow, s)

                mv = jnp.full((16,), mrow, jnp.float32)
                evecs = [jnp.exp(sv - mv) for sv in svecs]
                evecs[3] = jnp.where(lane < (LC - 48), evecs[3], 0.0)
                stot = jnp.sum(evecs[0] + evecs[1] + evecs[2] + evecs[3])
                sv_tot = jnp.full((16,), stot, jnp.float32)
                scv = [bf16r((ev / sv_tot) * jnp.float32(0.125))
                       for ev in evecs]

                att = [zero, zero, zero, zero]
                for c in range(LC):
                    w = jnp.full((16,), scv[c // 16][c % 16], jnp.float32)
                    x = [ctx_v[slot, s2 * LCP + c, pl.ds(l * 16, 16)]
                         for l in range(4)]
                    for l in range(4):
                        att[l] = att[l] + w * bf16r(x[l])

                u = jnp.sum(att[0] * t0v[0] + att[1] * t0v[1]
                            + att[2] * t0v[2] + att[3] * t0v[3])
                v = jnp.sum(att[0] * att[0] + att[1] * att[1]
                            + att[2] * att[2] + att[3] * att[3])
                lsel = lane == (jj % 16)
                u16 = jnp.where(lsel, u, u16)
                v16 = jnp.where(lsel, v, v16)
                for l in range(4):
                    pc_v[0, pl.ds(l * 16, 16)] = (
                        pc_v[0, pl.ds(l * 16, 16)] + att[l])

        u16c, v16c = u16, v16

        @pl.when(g % 4 == 3)
        def _():
            off = pl.multiple_of((g // 4) * 16, 16)
            uv_v[0, pl.ds(off, 16)] = u16c
            uv_v[1, pl.ds(off, 16)] = v16c

        return (u16, v16)

    pltpu.sync_copy(pc_v, pc_out.at[pl.ds(wid, 1), :])
    pltpu.sync_copy(uv_v, uv_out.at[:, pl.ds(base, RPW)])

    @pl.when(wid == 0)
    def _():
        pltpu.sync_copy(t0r_v.at[pl.ds(0, 1), :], t0_out)


def _attend(cpad, tgtids, ecw, etw):
    mesh = plsc.VectorSubcoreMesh(
        core_axis_name="c", subcore_axis_name="s",
        num_cores=NC, num_subcores=NS)
    f = pl.kernel(
        _attend_kernel,
        out_type=(
            jax.ShapeDtypeStruct((NW, D), jnp.float32),
            jax.ShapeDtypeStruct((4, B), jnp.float32),
            jax.ShapeDtypeStruct((1, D), jnp.float32),
        ),
        mesh=mesh,
        scratch_types=[
            pltpu.VMEM((RPW * LCP,), jnp.int32),
            pltpu.VMEM((RPW,), jnp.int32),
            pltpu.VMEM((8,), jnp.int32),
            pltpu.VMEM((2, IPC, D), jnp.float32),
            pltpu.VMEM((RPW, D), jnp.float32),
            pltpu.VMEM((8, D), jnp.float32),
            pltpu.VMEM((4, RPW), jnp.float32),
            pltpu.VMEM((1, D), jnp.float32),
            pltpu.SemaphoreType.DMA,
            pltpu.SemaphoreType.DMA,
            pltpu.SemaphoreType.DMA,
        ],
        compiler_params=pltpu.CompilerParams(
            needs_layout_passes=False, use_tc_tiling_on_sc=False),
        name="protonet_attend_sc",
    )
    return f(cpad, tgtids, ecw, etw)


def _mem_kernel(proto_ref, pcp_ref, uv_ref, t0_ref,
                copy_ref, sim2_ref, loss_ref, pcv_ref,
                sim2_s, acc_s, stat_s):
    i = pl.program_id(0)

    @pl.when(i == 0)
    def _():
        pcv = jnp.sum(pcp_ref[...], axis=0, keepdims=True) * (1.0 / B)
        pcv_ref[...] = pcv
        stat_s[0] = -jnp.inf
        stat_s[1] = jnp.float32(0.0)
        stat_s[2] = -jnp.inf
        stat_s[3] = jnp.float32(0.0)
        acc_s[...] = jnp.zeros((2, D), jnp.float32)

    P = proto_ref[...]
    copy_ref[...] = P
    pc = pcv_ref[...]
    sv = lax.dot_general(pc.astype(jnp.bfloat16), P.astype(jnp.bfloat16),
                         (((1,), (1,)), ((), ())),
                         preferred_element_type=jnp.float32)
    sim2_s[pl.ds(i, 1), :] = sv

    def flash(sval, srow, arow):
        m_old = stat_s[srow]
        s_old = stat_s[srow + 1]
        bm = jnp.max(sval)
        m_new = jnp.maximum(m_old, bm)
        c = jnp.exp(m_old - m_new)
        e = jnp.exp(sval - m_new)
        wsum = lax.dot_general(e, P, (((1,), (0,)), ((), ())),
                               precision=lax.Precision.HIGHEST,
                               preferred_element_type=jnp.float32)
        stat_s[srow] = m_new
        stat_s[srow + 1] = s_old * c + jnp.sum(e)
        acc_s[pl.ds(arow, 1), :] = acc_s[pl.ds(arow, 1), :] * c + wsum

    flash(sv, 0, 0)
    flash(sv * jnp.float32(-1.0 / SCALE), 2, 1)

    @pl.when(i == NBLK - 1)
    def _():
        sim2_ref[...] = sim2_s[...]
        s1 = stat_s[1]
        s2 = stat_s[3]
        pcv = pcv_ref[...]
        t0 = t0_ref[...]

        nb = jnp.maximum(jnp.sqrt(jnp.sum(t0 * t0)), 1e-8)

        def cosr(a):
            na = jnp.maximum(jnp.sqrt(jnp.sum(a * a)), 1e-8)
            return jnp.sum(a * t0) / (na * nb)

        mem_c = acc_s[0:1, :] * (jnp.float32(1.0 / SCALE) / s1)
        neg_c = acc_s[1:2, :] * (jnp.float32(1.0) / s2)

        u = uv_ref[0:1, :]
        v = uv_ref[1:2, :]
        na_att = jnp.maximum(jnp.sqrt(v), 1e-8)
        q = jnp.sum(u / (na_att * nb))

        conte = (q + cosr(pcv) + cosr(mem_c)) / jnp.float32(B + 2)
        negl = -cosr(neg_c)

        def ls(x):
            return -jnp.log(1.0 + jnp.exp(-x))

        loss_ref[...] = jnp.full((1, 1), -(ls(conte) + ls(negl)), jnp.float32)


def _memory_pass(proto, pc_parts, uv4, t0):
    return pl.pallas_call(
        _mem_kernel,
        grid=(NBLK,),
        in_specs=[
            pl.BlockSpec((RBLK, D), lambda i: (i, 0)),
            pl.BlockSpec((NW, D), lambda i: (0, 0)),
            pl.BlockSpec((4, B), lambda i: (0, 0)),
            pl.BlockSpec((1, D), lambda i: (0, 0)),
        ],
        out_specs=[
            pl.BlockSpec((RBLK, D), lambda i: (i, 0)),
            pl.BlockSpec((NBLK, RBLK), lambda i: (0, 0)),
            pl.BlockSpec((1, 1), lambda i: (0, 0)),
            pl.BlockSpec((1, D), lambda i: (0, 0)),
        ],
        out_shape=[
            jax.ShapeDtypeStruct((M, D), jnp.float32),
            jax.ShapeDtypeStruct((NBLK, RBLK), jnp.float32),
            jax.ShapeDtypeStruct((1, 1), jnp.float32),
            jax.ShapeDtypeStruct((1, D), jnp.float32),
        ],
        scratch_shapes=[
            pltpu.VMEM((NBLK, RBLK), jnp.float32),
            pltpu.VMEM((2, D), jnp.float32),
            pltpu.SMEM((8,), jnp.float32),
        ],
        compiler_params=pltpu.CompilerParams(
            dimension_semantics=("arbitrary",)),
        name="protonet_mem_tc",
    )(proto, pc_parts, uv4, t0)


def _scatter_kernel(tidx_ref, tval_ref, pcv_ref, repe_ref, mem_ref,
                    out_ref, rows_s, new_s, sem_g, sem_w):
    gets = []
    for k in range(K):
        r = tidx_ref[0, k]
        gets.append(pltpu.make_async_copy(
            mem_ref.at[pl.ds(r, 1), :], rows_s.at[pl.ds(k, 1), :], sem_g))
        gets[-1].start()
    for g in gets:
        g.wait()

    pc = pcv_ref[...]
    keep = repe_ref[0] > 0
    for k in range(K):
        old = rows_s[pl.ds(k, 1), :]
        summed = old + tval_ref[0, k] * pc
        nrm = jnp.maximum(jnp.sqrt(jnp.sum(summed * summed)), 1e-12)
        newrow = summed / nrm
        new_s[pl.ds(k, 1), :] = jnp.where(keep, newrow, old)

    puts = []
    for k in range(K):
        r = tidx_ref[0, k]
        puts.append(pltpu.make_async_copy(
            new_s.at[pl.ds(k, 1), :], out_ref.at[pl.ds(r, 1), :], sem_w))
        puts[-1].start()
    for p in puts:
        p.wait()


def _scatter_update(tidx, tval, pcv, repe, mem):
    return pl.pallas_call(
        _scatter_kernel,
        in_specs=[
            pl.BlockSpec(memory_space=pltpu.SMEM),
            pl.BlockSpec(memory_space=pltpu.SMEM),
            pl.BlockSpec((1, D), lambda: (0, 0)),
            pl.BlockSpec(memory_space=pltpu.SMEM),
            pl.BlockSpec(memory_space=pltpu.HBM),
        ],
        out_specs=pl.BlockSpec(memory_space=pltpu.HBM),
        out_shape=jax.ShapeDtypeStruct((M, D), jnp.float32),
        scratch_shapes=[
            pltpu.VMEM((K, D), jnp.float32),
            pltpu.VMEM((K, D), jnp.float32),
            pltpu.SemaphoreType.DMA,
            pltpu.SemaphoreType.DMA,
        ],
        input_output_aliases={4: 0},
        name="protonet_scatter_tc",
    )(tidx, tval, pcv, repe, mem)


def kernel(target, contex, embed_target_w, embed_contex_w, proto_memory,
           repe_end):
    tgt_flat = target.reshape(-1).astype(jnp.int32)
    cpad = jnp.concatenate(
        [contex, jnp.broadcast_to(contex[:, LC - 1:LC], (B, LCP - LC))],
        axis=1).astype(jnp.int32).reshape(-1)

    pc_parts, uv4, t0 = _attend(cpad, tgt_flat, embed_contex_w,
                                embed_target_w)
    mem_copy, sim2, loss, pcv = _memory_pass(
        proto_memory, pc_parts, uv4, t0)
    score2 = jax.nn.softmax(sim2.reshape(1, M), axis=-1) / SCALE
    top_sim, top_idx = lax.top_k(score2[0], K)
    repe = jnp.asarray(repe_end, jnp.int32).reshape(1)
    new_mem = _scatter_update(top_idx.reshape(1, K).astype(jnp.int32),
                              top_sim.reshape(1, K), pcv, repe, mem_copy)
    return loss.reshape(1), new_mem
